# baseline (speedup 1.0000x reference)
import jax
import jax.numpy as jnp
from jax.experimental import pallas as pl
from jax.experimental.pallas import tpu as pltpu

B, T, D = 32, 512, 128
WP, WF = 8, 8
NUM_REL = 4
NSPK = 2
K = WP + WF
R = 2
_OFFS = tuple(list(range(-WP, 0)) + list(range(1, WF + 1)))
_F32 = jnp.float32
_BF16 = jnp.bfloat16


def _down(a, s):
    t, w = a.shape
    return jnp.concatenate(
        [jnp.zeros((s, w), a.dtype), jax.lax.slice(a, (0, 0), (t - s, w))],
        axis=0)


def _up(a, s):
    t, w = a.shape
    return jnp.concatenate(
        [jax.lax.slice(a, (s, 0), (t, w)), jnp.zeros((s, w), a.dtype)],
        axis=0)


def _win_both(c):
    a = c + _down(c, 1)
    a = a + _down(a, 2)
    a = a + _down(a, 4)
    return _down(a, 1), _up(a, 8)


def _tree_sum(terms):
    while len(terms) > 1:
        terms = [terms[i] + terms[i + 1] for i in range(0, len(terms) - 1, 2)] \
            + ([terms[-1]] if len(terms) % 2 else [])
    return terms[0]


def _one_row(L, xb, qm, wrel_ref, wroot_ref, brg_ref, wq_ref, bq_ref,
             wk_ref, bk_ref, wv_ref, bv_ref, ws_ref, bs_ref,
             gamma_ref, beta_ref):
    t1 = jax.lax.broadcasted_iota(jnp.int32, (T, 1), 0)
    nvalid = t1 < L

    sp1 = qm[:, 1:2] > qm[:, 0:1]

    jv = nvalid.astype(_F32)
    sp1f = sp1.astype(_F32)
    xm = xb * jv
    c1 = (xm * sp1f).astype(_BF16)
    c0 = xm.astype(_BF16) - c1
    cnt = jnp.concatenate([jv - jv * sp1f, jv * sp1f], axis=1).astype(_BF16)

    S0p, S0f = _win_both(c0)
    S1p, S1f = _win_both(c1)
    Cp, Cf = _win_both(cnt)
    Cp, Cf = Cp.astype(_F32), Cf.astype(_F32)

    invp = 1.0 / jnp.maximum(Cp, 1.0)
    invf = 1.0 / jnp.maximum(Cf, 1.0)
    q0p = S0p.astype(_F32) * invp[:, 0:1]
    q1p = S1p.astype(_F32) * invp[:, 1:2]
    q0f = S0f.astype(_F32) * invf[:, 0:1]
    q1f = S1f.astype(_F32) * invf[:, 1:2]

    means = (jnp.where(sp1, q1p, q0p), jnp.where(sp1, q1f, q0f),
             jnp.where(sp1, q0p, q1p), jnp.where(sp1, q0f, q1f))

    hterms = [jnp.dot(xb, wroot_ref[...], preferred_element_type=_F32)]
    for r in range(NUM_REL):
        hterms.append(jnp.dot(means[r], wrel_ref[r],
                              preferred_element_type=_F32))
    ho = _tree_sum(hterms) + brg_ref[0]

    q = jnp.dot(ho, wq_ref[...], preferred_element_type=_F32) + bq_ref[0]
    k = jnp.dot(ho, wk_ref[...], preferred_element_type=_F32) + bk_ref[0]
    v = jnp.dot(ho, wv_ref[...], preferred_element_type=_F32) + bv_ref[0]

    col_iota = jax.lax.broadcasted_iota(jnp.int32, (D, K), 1)
    qb = q.astype(_BF16)
    kb = k.astype(_BF16)
    scs = [jnp.zeros((T, K), _F32) for _ in range(4)]
    for t, o in enumerate(_OFFS):
        k_o = pltpu.roll(kb, (-o) % T, 0)
        oh = (col_iota == t).astype(_BF16)
        scs[t % 4] = scs[t % 4] + jnp.dot(qb * k_o, oh,
                                          preferred_element_type=_F32)
    sc = _tree_sum(scs)

    lane = jax.lax.broadcasted_iota(jnp.int32, (T, K), 1)
    off_l = jnp.where(lane < WP, lane - WP, lane - (WP - 1))
    jo = jax.lax.broadcasted_iota(jnp.int32, (T, K), 0) + off_l
    valid = (jo >= 0) & (jo < L)
    sm = jnp.where(valid, sc, _F32(-1e30))
    m = jnp.max(sm, axis=1, keepdims=True)
    msafe = jnp.where(m > _F32(-0.5e30), m, 0.0)
    e = jnp.exp(sm - msafe)
    den_a = jnp.sum(e, axis=1, keepdims=True)
    en = e / jnp.maximum(den_a, 1e-16)

    enb = en.astype(_BF16)
    vb = v.astype(_BF16)
    ats = [jnp.zeros((T, D), _BF16) for _ in range(4)]
    for t, o in enumerate(_OFFS):
        ats[t % 4] = ats[t % 4] + enb[:, t:t + 1] * pltpu.roll(vb, (-o) % T, 0)
    attn = _tree_sum(ats).astype(_F32)

    h = attn + jnp.dot(ho, ws_ref[...], preferred_element_type=_F32)
    h = h + bs_ref[0]
    h = jnp.where(h >= 0, h, 0.01 * h)

    outp = jnp.where(nvalid, h, xb)
    y = xb + outp
    J = jnp.ones((D, D), _F32)
    mub = jnp.dot(y, J, preferred_element_type=_F32) * _F32(1.0 / D)
    yc = y - mub
    varb = jnp.dot(yc * yc, J, preferred_element_type=_F32) * _F32(1.0 / D)
    return yc * jax.lax.rsqrt(varb + 1e-5) * gamma_ref[0] + beta_ref[0]


def _row_kernel(dl_ref, x_ref, qm_ref, wrel_ref, wroot_ref, brg_ref,
                wq_ref, bq_ref, wk_ref, bk_ref, wv_ref, bv_ref,
                ws_ref, bs_ref, gamma_ref, beta_ref, out_ref):
    g = pl.program_id(0)
    for r in range(R):
        out_ref[r] = _one_row(
            dl_ref[R * g + r], x_ref[r], qm_ref[r], wrel_ref, wroot_ref,
            brg_ref, wq_ref, bq_ref, wk_ref, bk_ref, wv_ref, bv_ref,
            ws_ref, bs_ref, gamma_ref, beta_ref)


def kernel(x, qmask, dia_len, W_rel, W_root, b_rgcn, Wq, bq, Wk, bk,
           Wv, bv, Wskip, bskip, gamma, beta, interpret=False):
    row = lambda a: a.reshape(1, D)
    full = pl.BlockSpec((D, D), lambda b: (0, 0))
    vec = pl.BlockSpec((1, D), lambda b: (0, 0))
    out = pl.pallas_call(
        _row_kernel,
        grid=(B // R,),
        in_specs=[
            pl.BlockSpec(memory_space=pltpu.SMEM),
            pl.BlockSpec((R, T, D), lambda b: (b, 0, 0)),
            pl.BlockSpec((R, T, NSPK), lambda b: (b, 0, 0)),
            pl.BlockSpec((NUM_REL, D, D), lambda b: (0, 0, 0)),
            full, vec,
            full, vec, full, vec, full, vec,
            full, vec,
            vec, vec,
        ],
        out_specs=pl.BlockSpec((R, T, D), lambda b: (b, 0, 0)),
        out_shape=jax.ShapeDtypeStruct((B, T, D), jnp.float32),
        compiler_params=pltpu.CompilerParams(
            dimension_semantics=("arbitrary",)),
        interpret=interpret,
    )(dia_len.astype(jnp.int32), x, qmask, W_rel, W_root, row(b_rgcn),
      Wq * (1.0 / float(D) ** 0.5), row(bq * (1.0 / float(D) ** 0.5)),
      Wk, row(bk), Wv, row(bv), Wskip, row(bskip),
      row(gamma), row(beta))
    return (out, jnp.asarray(0.0, x.dtype))


# speedup vs baseline: 1.0424x; 1.0424x over previous
import jax
import jax.numpy as jnp
from jax.experimental import pallas as pl
from jax.experimental.pallas import tpu as pltpu

B, T, D = 32, 512, 128
WP, WF = 8, 8
NUM_REL = 4
NSPK = 2
K = WP + WF
R = 2
RT = R * T

_OFFS = tuple(list(range(-WP, 0)) + list(range(1, WF + 1)))
_F32 = jnp.float32
_BF16 = jnp.bfloat16


def _down(a, s):
    r, t, w = a.shape
    return jnp.concatenate(
        [jnp.zeros((r, s, w), a.dtype),
         jax.lax.slice(a, (0, 0, 0), (r, t - s, w))], axis=1)


def _up(a, s):
    r, t, w = a.shape
    return jnp.concatenate(
        [jax.lax.slice(a, (0, s, 0), (r, t, w)),
         jnp.zeros((r, s, w), a.dtype)], axis=1)


def _win_both(c):
    a = c + _down(c, 1)
    a = a + _down(a, 2)
    a = a + _down(a, 4)
    return _down(a, 1), _up(a, 8)


def _tree_sum(terms):
    while len(terms) > 1:
        terms = [terms[i] + terms[i + 1] for i in range(0, len(terms) - 1, 2)] \
            + ([terms[-1]] if len(terms) % 2 else [])
    return terms[0]


def _row_kernel(dl_ref, x_ref, qm_ref, wrel_ref, wroot_ref, brg_ref,
                wq_ref, bq_ref, wk_ref, bk_ref, wv_ref, bv_ref,
                ws_ref, bs_ref, gamma_ref, beta_ref, out_ref):
    g = pl.program_id(0)
    x3 = x_ref[...]
    qm = qm_ref[...].reshape(RT, NSPK)
    r_iota = jax.lax.broadcasted_iota(jnp.int32, (R, T, 1), 0)
    L3 = jnp.full((R, T, 1), dl_ref[R * g], jnp.int32)
    for r in range(1, R):
        L3 = jnp.where(r_iota == r, dl_ref[R * g + r], L3)
    t3 = jax.lax.broadcasted_iota(jnp.int32, (R, T, 1), 1)
    nvalid3 = t3 < L3
    xf = x3.reshape(RT, D)
    nvalid = nvalid3.reshape(RT, 1)
    Lf = L3.reshape(RT, 1)

    sp1 = qm[:, 1:2] > qm[:, 0:1]

    jv = nvalid.astype(_F32)
    sp1f = sp1.astype(_F32)
    xm = xf * jv
    c1 = (xm * sp1f).astype(_BF16).reshape(R, T, D)
    c0 = xm.astype(_BF16).reshape(R, T, D) - c1
    cnt = jnp.concatenate([jv - jv * sp1f, jv * sp1f],
                          axis=1).astype(_BF16).reshape(R, T, NSPK)

    S0p, S0f = _win_both(c0)
    S1p, S1f = _win_both(c1)
    Cp, Cf = _win_both(cnt)
    S0p, S0f = S0p.reshape(RT, D), S0f.reshape(RT, D)
    S1p, S1f = S1p.reshape(RT, D), S1f.reshape(RT, D)
    Cp = Cp.reshape(RT, NSPK).astype(_F32)
    Cf = Cf.reshape(RT, NSPK).astype(_F32)

    invp = 1.0 / jnp.maximum(Cp, 1.0)
    invf = 1.0 / jnp.maximum(Cf, 1.0)
    q0p = S0p.astype(_F32) * invp[:, 0:1]
    q1p = S1p.astype(_F32) * invp[:, 1:2]
    q0f = S0f.astype(_F32) * invf[:, 0:1]
    q1f = S1f.astype(_F32) * invf[:, 1:2]

    means = (jnp.where(sp1, q1p, q0p), jnp.where(sp1, q1f, q0f),
             jnp.where(sp1, q0p, q1p), jnp.where(sp1, q0f, q1f))

    hterms = [jnp.dot(xf, wroot_ref[...], preferred_element_type=_F32)]
    for r in range(NUM_REL):
        hterms.append(jnp.dot(means[r], wrel_ref[r],
                              preferred_element_type=_F32))
    ho = _tree_sum(hterms) + brg_ref[0]

    q = jnp.dot(ho, wq_ref[...], preferred_element_type=_F32) + bq_ref[0]
    k = jnp.dot(ho, wk_ref[...], preferred_element_type=_F32) + bk_ref[0]
    v = jnp.dot(ho, wv_ref[...], preferred_element_type=_F32) + bv_ref[0]

    col_iota = jax.lax.broadcasted_iota(jnp.int32, (D, K), 1)
    qb = q.astype(_BF16)
    kb = k.astype(_BF16)
    scs = [jnp.zeros((RT, K), _F32) for _ in range(4)]
    for t, o in enumerate(_OFFS):
        k_o = pltpu.roll(kb, (-o) % RT, 0)
        oh = (col_iota == t).astype(_BF16)
        scs[t % 4] = scs[t % 4] + jnp.dot(qb * k_o, oh,
                                          preferred_element_type=_F32)
    sc = _tree_sum(scs)

    lane = jax.lax.broadcasted_iota(jnp.int32, (RT, K), 1)
    off_l = jnp.where(lane < WP, lane - WP, lane - (WP - 1))
    tf = jax.lax.broadcasted_iota(jnp.int32, (R, T, K), 1).reshape(RT, K)
    jo = tf + off_l
    valid = (jo >= 0) & (jo < Lf)
    sm = jnp.where(valid, sc, _F32(-1e30))
    m = jnp.max(sm, axis=1, keepdims=True)
    msafe = jnp.where(m > _F32(-0.5e30), m, 0.0)
    e = jnp.exp(sm - msafe)
    den_a = jnp.sum(e, axis=1, keepdims=True)
    en = e / jnp.maximum(den_a, 1e-16)

    enb = en.astype(_BF16)
    vb = v.astype(_BF16)
    ats = [jnp.zeros((RT, D), _BF16) for _ in range(4)]
    for t, o in enumerate(_OFFS):
        ats[t % 4] = ats[t % 4] + enb[:, t:t + 1] * pltpu.roll(vb, (-o) % RT, 0)
    attn = _tree_sum(ats).astype(_F32)

    h = attn + jnp.dot(ho, ws_ref[...], preferred_element_type=_F32)
    h = h + bs_ref[0]
    h = jnp.where(h >= 0, h, 0.01 * h)

    outp = jnp.where(nvalid, h, xf)
    y = xf + outp
    J = jnp.ones((D, D), _F32)
    mub = jnp.dot(y, J, preferred_element_type=_F32) * _F32(1.0 / D)
    yc = y - mub
    varb = jnp.dot(yc * yc, J, preferred_element_type=_F32) * _F32(1.0 / D)
    out = yc * jax.lax.rsqrt(varb + 1e-5) * gamma_ref[0] + beta_ref[0]
    out_ref[...] = out.reshape(R, T, D)


def kernel(x, qmask, dia_len, W_rel, W_root, b_rgcn, Wq, bq, Wk, bk,
           Wv, bv, Wskip, bskip, gamma, beta, interpret=False):
    row = lambda a: a.reshape(1, D)
    full = pl.BlockSpec((D, D), lambda b: (0, 0))
    vec = pl.BlockSpec((1, D), lambda b: (0, 0))
    out = pl.pallas_call(
        _row_kernel,
        grid=(B // R,),
        in_specs=[
            pl.BlockSpec(memory_space=pltpu.SMEM),
            pl.BlockSpec((R, T, D), lambda b: (b, 0, 0)),
            pl.BlockSpec((R, T, NSPK), lambda b: (b, 0, 0)),
            pl.BlockSpec((NUM_REL, D, D), lambda b: (0, 0, 0)),
            full, vec,
            full, vec, full, vec, full, vec,
            full, vec,
            vec, vec,
        ],
        out_specs=pl.BlockSpec((R, T, D), lambda b: (b, 0, 0)),
        out_shape=jax.ShapeDtypeStruct((B, T, D), jnp.float32),
        compiler_params=pltpu.CompilerParams(
            dimension_semantics=("arbitrary",)),
        interpret=interpret,
    )(dia_len.astype(jnp.int32), x, qmask, W_rel, W_root, row(b_rgcn),
      Wq * (1.0 / float(D) ** 0.5), row(bq * (1.0 / float(D) ** 0.5)),
      Wk, row(bk), Wv, row(bv), Wskip, row(bskip),
      row(gamma), row(beta))
    return (out, jnp.asarray(0.0, x.dtype))


# speedup vs baseline: 1.0447x; 1.0022x over previous
import jax
import jax.numpy as jnp
from jax.experimental import pallas as pl
from jax.experimental.pallas import tpu as pltpu

B, T, D = 32, 512, 128
WP, WF = 8, 8
NUM_REL = 4
NSPK = 2
K = WP + WF
R = 2
RT = R * T

_OFFS = tuple(list(range(-WP, 0)) + list(range(1, WF + 1)))
_F32 = jnp.float32
_BF16 = jnp.bfloat16


def _down(a, s):
    r, t, w = a.shape
    return jnp.concatenate(
        [jnp.zeros((r, s, w), a.dtype),
         jax.lax.slice(a, (0, 0, 0), (r, t - s, w))], axis=1)


def _up(a, s):
    r, t, w = a.shape
    return jnp.concatenate(
        [jax.lax.slice(a, (0, s, 0), (r, t, w)),
         jnp.zeros((r, s, w), a.dtype)], axis=1)


def _win_both(c):
    a = c + _down(c, 1)
    a = a + _down(a, 2)
    a = a + _down(a, 4)
    return _down(a, 1), _up(a, 8)


def _tree_sum(terms):
    while len(terms) > 1:
        terms = [terms[i] + terms[i + 1] for i in range(0, len(terms) - 1, 2)] \
            + ([terms[-1]] if len(terms) % 2 else [])
    return terms[0]


def _row_kernel(dl_ref, x_ref, qm_ref, wrel_ref, wroot_ref, brg_ref,
                wq_ref, bq_ref, wk_ref, bk_ref, wv_ref, bv_ref,
                ws_ref, bs_ref, gamma_ref, beta_ref, out_ref):
    g = pl.program_id(0)
    x3 = x_ref[...]
    qm = qm_ref[...].reshape(RT, NSPK)
    r_iota = jax.lax.broadcasted_iota(jnp.int32, (R, T, 1), 0)
    L3 = jnp.full((R, T, 1), dl_ref[R * g], jnp.int32)
    for r in range(1, R):
        L3 = jnp.where(r_iota == r, dl_ref[R * g + r], L3)
    t3 = jax.lax.broadcasted_iota(jnp.int32, (R, T, 1), 1)
    nvalid3 = t3 < L3
    xf = x3.reshape(RT, D)
    nvalid = nvalid3.reshape(RT, 1)
    Lf = L3.reshape(RT, 1)

    sp1 = qm[:, 1:2] > qm[:, 0:1]

    jv = nvalid.astype(_F32)
    sp1f = sp1.astype(_F32)
    xm = xf * jv
    c1 = (xm * sp1f).astype(_BF16)
    c0 = xm.astype(_BF16) - c1
    c01 = jnp.concatenate([c0, c1], axis=1).reshape(R, T, 2 * D)
    cnt = jnp.concatenate([jv - jv * sp1f, jv * sp1f],
                          axis=1).astype(_BF16).reshape(R, T, NSPK)

    Sp, Sf = _win_both(c01)
    Cp, Cf = _win_both(cnt)
    Sp, Sf = Sp.reshape(RT, 2 * D), Sf.reshape(RT, 2 * D)
    S0p, S1p = Sp[:, :D], Sp[:, D:]
    S0f, S1f = Sf[:, :D], Sf[:, D:]
    Cp = Cp.reshape(RT, NSPK).astype(_F32)
    Cf = Cf.reshape(RT, NSPK).astype(_F32)

    invp = 1.0 / jnp.maximum(Cp, 1.0)
    invf = 1.0 / jnp.maximum(Cf, 1.0)
    q0p = S0p.astype(_F32) * invp[:, 0:1]
    q1p = S1p.astype(_F32) * invp[:, 1:2]
    q0f = S0f.astype(_F32) * invf[:, 0:1]
    q1f = S1f.astype(_F32) * invf[:, 1:2]

    means = (jnp.where(sp1, q1p, q0p), jnp.where(sp1, q1f, q0f),
             jnp.where(sp1, q0p, q1p), jnp.where(sp1, q0f, q1f))

    hterms = [jnp.dot(xf, wroot_ref[...], preferred_element_type=_F32)]
    for r in range(NUM_REL):
        hterms.append(jnp.dot(means[r], wrel_ref[r],
                              preferred_element_type=_F32))
    ho = _tree_sum(hterms) + brg_ref[0]

    q = jnp.dot(ho, wq_ref[...], preferred_element_type=_F32) + bq_ref[0]
    k = jnp.dot(ho, wk_ref[...], preferred_element_type=_F32) + bk_ref[0]
    v = jnp.dot(ho, wv_ref[...], preferred_element_type=_F32) + bv_ref[0]

    col_iota = jax.lax.broadcasted_iota(jnp.int32, (D, K), 1)
    qb = q.astype(_BF16)
    kb = k.astype(_BF16)
    scs = [jnp.zeros((RT, K), _F32) for _ in range(4)]
    for t, o in enumerate(_OFFS):
        k_o = pltpu.roll(kb, (-o) % RT, 0)
        oh = (col_iota == t).astype(_BF16)
        scs[t % 4] = scs[t % 4] + jnp.dot(qb * k_o, oh,
                                          preferred_element_type=_F32)
    sc = _tree_sum(scs)

    lane = jax.lax.broadcasted_iota(jnp.int32, (RT, K), 1)
    off_l = jnp.where(lane < WP, lane - WP, lane - (WP - 1))
    tf = jax.lax.broadcasted_iota(jnp.int32, (R, T, K), 1).reshape(RT, K)
    jo = tf + off_l
    valid = (jo >= 0) & (jo < Lf)
    sm = jnp.where(valid, sc, _F32(-1e30))
    m = jnp.max(sm, axis=1, keepdims=True)
    msafe = jnp.where(m > _F32(-0.5e30), m, 0.0)
    e = jnp.exp(sm - msafe)
    den_a = jnp.sum(e, axis=1, keepdims=True)
    en = e / jnp.maximum(den_a, 1e-16)

    enb = en.astype(_BF16)
    vb = v.astype(_BF16)
    ats = [jnp.zeros((RT, D), _BF16) for _ in range(4)]
    for t, o in enumerate(_OFFS):
        ats[t % 4] = ats[t % 4] + enb[:, t:t + 1] * pltpu.roll(vb, (-o) % RT, 0)
    attn = _tree_sum(ats).astype(_F32)

    h = attn + jnp.dot(ho, ws_ref[...], preferred_element_type=_F32)
    h = h + bs_ref[0]
    h = jnp.where(h >= 0, h, 0.01 * h)

    outp = jnp.where(nvalid, h, xf)
    y = xf + outp
    J = jnp.ones((D, D), _F32)
    mub = jnp.dot(y, J, preferred_element_type=_F32) * _F32(1.0 / D)
    yc = y - mub
    varb = jnp.dot(yc * yc, J, preferred_element_type=_F32) * _F32(1.0 / D)
    out = yc * jax.lax.rsqrt(varb + 1e-5) * gamma_ref[0] + beta_ref[0]
    out_ref[...] = out.reshape(R, T, D)


def kernel(x, qmask, dia_len, W_rel, W_root, b_rgcn, Wq, bq, Wk, bk,
           Wv, bv, Wskip, bskip, gamma, beta, interpret=False):
    row = lambda a: a.reshape(1, D)
    full = pl.BlockSpec((D, D), lambda b: (0, 0))
    vec = pl.BlockSpec((1, D), lambda b: (0, 0))
    out = pl.pallas_call(
        _row_kernel,
        grid=(B // R,),
        in_specs=[
            pl.BlockSpec(memory_space=pltpu.SMEM),
            pl.BlockSpec((R, T, D), lambda b: (b, 0, 0)),
            pl.BlockSpec((R, T, NSPK), lambda b: (b, 0, 0)),
            pl.BlockSpec((NUM_REL, D, D), lambda b: (0, 0, 0)),
            full, vec,
            full, vec, full, vec, full, vec,
            full, vec,
            vec, vec,
        ],
        out_specs=pl.BlockSpec((R, T, D), lambda b: (b, 0, 0)),
        out_shape=jax.ShapeDtypeStruct((B, T, D), jnp.float32),
        compiler_params=pltpu.CompilerParams(
            dimension_semantics=("arbitrary",)),
        interpret=interpret,
    )(dia_len.astype(jnp.int32), x, qmask, W_rel, W_root, row(b_rgcn),
      Wq * (1.0 / float(D) ** 0.5), row(bq * (1.0 / float(D) ** 0.5)),
      Wk, row(bk), Wv, row(bv), Wskip, row(bskip),
      row(gamma), row(beta))
    return (out, jnp.asarray(0.0, x.dtype))


# speedup vs baseline: 1.0448x; 1.0001x over previous
import jax
import jax.numpy as jnp
from jax.experimental import pallas as pl
from jax.experimental.pallas import tpu as pltpu

B, T, D = 32, 512, 128
WP, WF = 8, 8
NUM_REL = 4
NSPK = 2
K = WP + WF
R = 2
RT = R * T

_OFFS = tuple(list(range(-WP, 0)) + list(range(1, WF + 1)))
_F32 = jnp.float32
_BF16 = jnp.bfloat16


def _down(a, s):
    r, t, w = a.shape
    return jnp.concatenate(
        [jnp.zeros((r, s, w), a.dtype),
         jax.lax.slice(a, (0, 0, 0), (r, t - s, w))], axis=1)


def _up(a, s):
    r, t, w = a.shape
    return jnp.concatenate(
        [jax.lax.slice(a, (0, s, 0), (r, t, w)),
         jnp.zeros((r, s, w), a.dtype)], axis=1)


def _win_both(c):
    a = c + _down(c, 1)
    a = a + _down(a, 2)
    a = a + _down(a, 4)
    return _down(a, 1), _up(a, 8)


def _tree_sum(terms):
    while len(terms) > 1:
        terms = [terms[i] + terms[i + 1] for i in range(0, len(terms) - 1, 2)] \
            + ([terms[-1]] if len(terms) % 2 else [])
    return terms[0]


def _row_kernel(dl_ref, x_ref, qm_ref, wrel_ref, wroot_ref, brg_ref,
                wq_ref, bq_ref, wk_ref, bk_ref, wv_ref, bv_ref,
                ws_ref, bs_ref, gamma_ref, beta_ref, out_ref):
    g = pl.program_id(0)
    x3 = x_ref[...]
    qm = qm_ref[...].reshape(RT, NSPK)
    r_iota = jax.lax.broadcasted_iota(jnp.int32, (R, T, 1), 0)
    L3 = jnp.full((R, T, 1), dl_ref[R * g], jnp.int32)
    for r in range(1, R):
        L3 = jnp.where(r_iota == r, dl_ref[R * g + r], L3)
    t3 = jax.lax.broadcasted_iota(jnp.int32, (R, T, 1), 1)
    nvalid3 = t3 < L3
    xf = x3.reshape(RT, D)
    nvalid = nvalid3.reshape(RT, 1)
    Lf = L3.reshape(RT, 1)

    sp1 = qm[:, 1:2] > qm[:, 0:1]

    jv = nvalid.astype(_F32)
    sp1f = sp1.astype(_F32)
    xm = xf * jv
    c1 = (xm * sp1f).astype(_BF16)
    c0 = xm.astype(_BF16) - c1
    c01 = jnp.concatenate([c0, c1], axis=1).reshape(R, T, 2 * D)
    cnt = jnp.concatenate([jv - jv * sp1f, jv * sp1f],
                          axis=1).astype(_BF16).reshape(R, T, NSPK)

    Sp, Sf = _win_both(c01)
    Cp, Cf = _win_both(cnt)
    Sp, Sf = Sp.reshape(RT, 2 * D), Sf.reshape(RT, 2 * D)
    S0p, S1p = Sp[:, :D], Sp[:, D:]
    S0f, S1f = Sf[:, :D], Sf[:, D:]
    Cp = Cp.reshape(RT, NSPK).astype(_F32)
    Cf = Cf.reshape(RT, NSPK).astype(_F32)

    invp = 1.0 / jnp.maximum(Cp, 1.0)
    invf = 1.0 / jnp.maximum(Cf, 1.0)
    q0p = S0p.astype(_F32) * invp[:, 0:1]
    q1p = S1p.astype(_F32) * invp[:, 1:2]
    q0f = S0f.astype(_F32) * invf[:, 0:1]
    q1f = S1f.astype(_F32) * invf[:, 1:2]

    means = (jnp.where(sp1, q1p, q0p), jnp.where(sp1, q1f, q0f),
             jnp.where(sp1, q0p, q1p), jnp.where(sp1, q0f, q1f))

    hterms = [jnp.dot(xf, wroot_ref[...], preferred_element_type=_F32)]
    for r in range(NUM_REL):
        hterms.append(jnp.dot(means[r], wrel_ref[r],
                              preferred_element_type=_F32))
    ho = _tree_sum(hterms) + brg_ref[0]

    q = jnp.dot(ho, wq_ref[...], preferred_element_type=_F32) + bq_ref[0]
    k = jnp.dot(ho, wk_ref[...], preferred_element_type=_F32) + bk_ref[0]
    v = jnp.dot(ho, wv_ref[...], preferred_element_type=_F32) + bv_ref[0]

    col_iota = jax.lax.broadcasted_iota(jnp.int32, (D, K), 1)
    qb = q.astype(_BF16)
    kb = k.astype(_BF16)
    scs = [jnp.zeros((RT, K), _F32) for _ in range(4)]
    for t, o in enumerate(_OFFS):
        k_o = pltpu.roll(kb, (-o) % RT, 0)
        oh = (col_iota == t).astype(_BF16)
        scs[t % 4] = scs[t % 4] + jnp.dot(qb * k_o, oh,
                                          preferred_element_type=_F32)
    sc = _tree_sum(scs)

    lane = jax.lax.broadcasted_iota(jnp.int32, (RT, K), 1)
    off_l = jnp.where(lane < WP, lane - WP, lane - (WP - 1))
    tf = jax.lax.broadcasted_iota(jnp.int32, (R, T, K), 1).reshape(RT, K)
    jo = tf + off_l
    valid = (jo >= 0) & (jo < Lf)
    sm = jnp.where(valid, sc, _F32(-1e30))
    m = jnp.max(sm, axis=1, keepdims=True)
    msafe = jnp.where(m > _F32(-0.5e30), m, 0.0)
    e = jnp.exp(sm - msafe)
    den_a = jnp.sum(e, axis=1, keepdims=True)
    en = e / jnp.maximum(den_a, 1e-16)

    enb = en.astype(_BF16)
    vb = v.astype(_BF16)
    ats = [jnp.zeros((RT, D), _BF16) for _ in range(4)]
    for t, o in enumerate(_OFFS):
        ats[t % 4] = ats[t % 4] + enb[:, t:t + 1] * pltpu.roll(vb, (-o) % RT, 0)
    attn = _tree_sum(ats).astype(_F32)

    h = attn + jnp.dot(ho, ws_ref[...], preferred_element_type=_F32)
    h = h + bs_ref[0]
    h = jnp.where(h >= 0, h, 0.01 * h)

    outp = jnp.where(nvalid, h, xf)
    y = xf + outp
    J = jnp.ones((D, D), _F32)
    mub = jnp.dot(y, J, preferred_element_type=_F32) * _F32(1.0 / D)
    yc = y - mub
    varb = jnp.dot(yc * yc, J, preferred_element_type=_F32) * _F32(1.0 / D)
    out = yc * jax.lax.rsqrt(varb + 1e-5) * gamma_ref[0] + beta_ref[0]
    out_ref[...] = out.reshape(R, T, D)


def kernel(x, qmask, dia_len, W_rel, W_root, b_rgcn, Wq, bq, Wk, bk,
           Wv, bv, Wskip, bskip, gamma, beta):
    row = lambda a: a.reshape(1, D)
    full = pl.BlockSpec((D, D), lambda b: (0, 0))
    vec = pl.BlockSpec((1, D), lambda b: (0, 0))
    out = pl.pallas_call(
        _row_kernel,
        grid=(B // R,),
        in_specs=[
            pl.BlockSpec(memory_space=pltpu.SMEM),
            pl.BlockSpec((R, T, D), lambda b: (b, 0, 0)),
            pl.BlockSpec((R, T, NSPK), lambda b: (b, 0, 0)),
            pl.BlockSpec((NUM_REL, D, D), lambda b: (0, 0, 0)),
            full, vec,
            full, vec, full, vec, full, vec,
            full, vec,
            vec, vec,
        ],
        out_specs=pl.BlockSpec((R, T, D), lambda b: (b, 0, 0)),
        out_shape=jax.ShapeDtypeStruct((B, T, D), jnp.float32),
        compiler_params=pltpu.CompilerParams(
            dimension_semantics=("arbitrary",)),
    )(dia_len.astype(jnp.int32), x, qmask, W_rel, W_root, row(b_rgcn),
      Wq * (1.0 / float(D) ** 0.5), row(bq * (1.0 / float(D) ** 0.5)),
      Wk, row(bk), Wv, row(bv), Wskip, row(bskip),
      row(gamma), row(beta))
    return (out, jnp.asarray(0.0, x.dtype))
